# baseline (speedup 1.0000x reference)
import functools

import jax
import jax.numpy as jnp
from jax import lax
from jax.experimental import pallas as pl
from jax.experimental.pallas import tpu as pltpu
from jax.experimental.pallas import tpu_sc as plsc

NUM_ACT = 2048
SIZE = 51
RANGE_ = 4.0
GRID = 2.0 * RANGE_ / (SIZE - 1)
HALF = SIZE // 2
TABLE = NUM_ACT * SIZE

NC, NS, L = 2, 16, 16
NW = NC * NS
ROWS_PER_CHUNK = 2
CHUNK = ROWS_PER_CHUNK * NUM_ACT


def _project_body(cs_ref, out_ref):
    cs = cs_ref[...]
    slopes = jnp.clip(cs[:, 1:] - cs[:, :-1], 0.0, jnp.float32(GRID))
    k = lax.broadcasted_iota(jnp.int32, (SIZE - 1, SIZE), 0)
    j = lax.broadcasted_iota(jnp.int32, (SIZE - 1, SIZE), 1)
    m = (j > k).astype(jnp.float32)
    cum = lax.dot_general(
        slopes, m, (((1,), (0,)), ((), ())),
        preferred_element_type=jnp.float32,
        precision=lax.Precision.HIGHEST)
    out_ref[...] = cum - cum[:, HALF:HALF + 1]


def _make_sc_kernel(n_elems):
    chunks_total = n_elems // CHUNK
    ch_per_w = chunks_total // NW
    mesh = plsc.VectorSubcoreMesh(
        core_axis_name="c", subcore_axis_name="s",
        num_cores=NC, num_subcores=NS)

    @functools.partial(
        pl.kernel,
        out_type=jax.ShapeDtypeStruct((n_elems,), jnp.float32),
        mesh=mesh,
        compiler_params=pltpu.CompilerParams(needs_layout_passes=False),
        scratch_types=[
            pltpu.VMEM((TABLE,), jnp.float32),
            pltpu.VMEM((NUM_ACT,), jnp.float32),
            pltpu.VMEM((NUM_ACT,), jnp.float32),
            pltpu.VMEM((CHUNK,), jnp.float32),
            pltpu.VMEM((CHUNK,), jnp.float32),
            pltpu.VMEM((CHUNK,), jnp.float32),
            pltpu.VMEM((CHUNK,), jnp.float32),
            pltpu.SemaphoreType.DMA,
            pltpu.SemaphoreType.DMA,
            pltpu.SemaphoreType.DMA,
            pltpu.SemaphoreType.DMA,
        ],
    )
    def sc_kernel(x_hbm, cv_hbm, pm_hbm, inv_hbm, out_hbm,
                  table, pm_v, inv_v, xb0, xb1, ob0, ob1,
                  sin0, sin1, sout0, sout1):
        wid = lax.axis_index("s") * NC + lax.axis_index("c")
        pltpu.sync_copy(cv_hbm, table)
        pltpu.sync_copy(pm_hbm, pm_v)
        pltpu.sync_copy(inv_hbm, inv_v)
        base = wid * (ch_per_w * CHUNK)
        xbs = (xb0, xb1)
        obs = (ob0, ob1)
        sins = (sin0, sin1)
        souts = (sout0, sout1)

        def start_in(c, b):
            pltpu.async_copy(
                x_hbm.at[pl.ds(base + c * CHUNK, CHUNK)], xbs[b], sins[b])

        def start_out(c, b):
            pltpu.async_copy(
                obs[b], out_hbm.at[pl.ds(base + c * CHUNK, CHUNK)], souts[b])

        def wait_in(b):
            pltpu.make_async_copy(
                x_hbm.at[pl.ds(base, CHUNK)], xbs[b], sins[b]).wait()

        def wait_out(b):
            pltpu.make_async_copy(
                obs[b], out_hbm.at[pl.ds(base, CHUNK)], souts[b]).wait()

        iota51 = lax.iota(jnp.int32, L) * SIZE

        def compute(b):
            xb = xbs[b]
            ob = obs[b]

            @plsc.parallel_loop(0, NUM_ACT // L, unroll=8)
            def _(i):
                col = i * L
                pm = pm_v[pl.ds(col, L)]
                iv = inv_v[pl.ds(col, L)]
                chv = iota51 + col * SIZE
                for r in range(ROWS_PER_CHUNK):
                    o = r * NUM_ACT + col
                    xv = xb[pl.ds(o, L)]
                    u = xv * pm + jnp.float32(HALF)
                    uc = jnp.minimum(jnp.maximum(u, 0.0),
                                     jnp.float32(SIZE - 2))
                    fl = uc.astype(jnp.int32)
                    frac = u - fl.astype(jnp.float32)
                    idx = chv + fl
                    a = plsc.load_gather(table, [idx])
                    c2 = plsc.load_gather(table, [idx + 1])
                    ob[pl.ds(o, L)] = (a + (c2 - a) * frac) * iv

        start_in(0, 0)
        start_in(1, 1)

        @pl.loop(0, ch_per_w, step=2)
        def _(c):
            for b in range(2):
                cc = c + b
                wait_in(b)

                @pl.when(cc >= 2)
                def _():
                    wait_out(b)

                compute(b)
                start_out(cc, b)

                @pl.when(cc + 2 < ch_per_w)
                def _():
                    start_in(cc + 2, b)

        wait_out(0)
        wait_out(1)

    return sc_kernel


def kernel(input, coefficients_vect, scaling_coeffs_vect):
    b, c = input.shape
    cs = coefficients_vect.reshape(NUM_ACT, SIZE)
    cv = pl.pallas_call(
        _project_body,
        out_shape=jax.ShapeDtypeStruct((NUM_ACT, SIZE), jnp.float32),
    )(cs).reshape(-1)
    s = scaling_coeffs_vect.reshape(NUM_ACT)
    pm = s * jnp.float32(1.0 / GRID)
    inv = 1.0 / s
    out = _make_sc_kernel(b * c)(input.reshape(-1), cv, pm, inv)
    return out.reshape(b, c)


# speedup vs baseline: 1.1267x; 1.1267x over previous
import functools

import jax
import jax.numpy as jnp
from jax import lax
from jax.experimental import pallas as pl
from jax.experimental.pallas import tpu as pltpu
from jax.experimental.pallas import tpu_sc as plsc

NUM_ACT = 2048
SIZE = 51
RANGE_ = 4.0
GRID = 2.0 * RANGE_ / (SIZE - 1)
HALF = SIZE // 2
TABLE = NUM_ACT * SIZE

NC, NS, L = 2, 16, 16
NW = NC * NS
ROWS_PER_CHUNK = 2
CHUNK = ROWS_PER_CHUNK * NUM_ACT


def _project_body(cs_ref, out_ref):
    cs = cs_ref[...]
    slopes = jnp.clip(cs[1:, :] - cs[:-1, :], 0.0, jnp.float32(GRID))
    j = lax.broadcasted_iota(jnp.int32, (SIZE, SIZE - 1), 0)
    k = lax.broadcasted_iota(jnp.int32, (SIZE, SIZE - 1), 1)
    m = (j > k).astype(jnp.float32)
    cum = lax.dot_general(
        m, slopes, (((1,), (0,)), ((), ())),
        preferred_element_type=jnp.float32,
        precision=lax.Precision.HIGHEST)
    out_ref[...] = cum - cum[HALF:HALF + 1, :]


def _make_sc_kernel(n_elems):
    chunks_total = n_elems // CHUNK
    ch_per_w = chunks_total // NW
    mesh = plsc.VectorSubcoreMesh(
        core_axis_name="c", subcore_axis_name="s",
        num_cores=NC, num_subcores=NS)

    @functools.partial(
        pl.kernel,
        out_type=jax.ShapeDtypeStruct((n_elems,), jnp.float32),
        mesh=mesh,
        compiler_params=pltpu.CompilerParams(needs_layout_passes=False),
        scratch_types=[
            pltpu.VMEM((TABLE,), jnp.float32),
            pltpu.VMEM((NUM_ACT,), jnp.float32),
            pltpu.VMEM((NUM_ACT,), jnp.float32),
            pltpu.VMEM((CHUNK,), jnp.float32),
            pltpu.VMEM((CHUNK,), jnp.float32),
            pltpu.VMEM((CHUNK,), jnp.float32),
            pltpu.VMEM((CHUNK,), jnp.float32),
            pltpu.SemaphoreType.DMA,
            pltpu.SemaphoreType.DMA,
            pltpu.SemaphoreType.DMA,
            pltpu.SemaphoreType.DMA,
        ],
    )
    def sc_kernel(x_hbm, cv_hbm, pm_hbm, inv_hbm, out_hbm,
                  table, pm_v, inv_v, xb0, xb1, ob0, ob1,
                  sin0, sin1, sout0, sout1):
        wid = lax.axis_index("s") * NC + lax.axis_index("c")
        pltpu.sync_copy(cv_hbm, table)
        pltpu.sync_copy(pm_hbm, pm_v)
        pltpu.sync_copy(inv_hbm, inv_v)
        base = wid * (ch_per_w * CHUNK)
        xbs = (xb0, xb1)
        obs = (ob0, ob1)
        sins = (sin0, sin1)
        souts = (sout0, sout1)

        def start_in(c, b):
            pltpu.async_copy(
                x_hbm.at[pl.ds(base + c * CHUNK, CHUNK)], xbs[b], sins[b])

        def start_out(c, b):
            pltpu.async_copy(
                obs[b], out_hbm.at[pl.ds(base + c * CHUNK, CHUNK)], souts[b])

        def wait_in(b):
            pltpu.make_async_copy(
                x_hbm.at[pl.ds(base, CHUNK)], xbs[b], sins[b]).wait()

        def wait_out(b):
            pltpu.make_async_copy(
                obs[b], out_hbm.at[pl.ds(base, CHUNK)], souts[b]).wait()

        iota_l = lax.iota(jnp.int32, L)

        def compute(b):
            xb = xbs[b]
            ob = obs[b]

            @plsc.parallel_loop(0, NUM_ACT // L, unroll=4)
            def _(i):
                col = i * L
                pm = pm_v[pl.ds(col, L)]
                iv = inv_v[pl.ds(col, L)]
                chv = iota_l + col
                for r in range(ROWS_PER_CHUNK):
                    o = r * NUM_ACT + col
                    xv = xb[pl.ds(o, L)]
                    u = xv * pm + jnp.float32(HALF)
                    uc = jnp.minimum(jnp.maximum(u, 0.0),
                                     jnp.float32(SIZE - 2))
                    fl = uc.astype(jnp.int32)
                    frac = u - fl.astype(jnp.float32)
                    idx = (fl << 11) + chv
                    a = plsc.load_gather(table, [idx])
                    c2 = plsc.load_gather(table, [idx + NUM_ACT])
                    ob[pl.ds(o, L)] = (a + (c2 - a) * frac) * iv

        start_in(0, 0)
        start_in(1, 1)

        @pl.loop(0, ch_per_w, step=2)
        def _(c):
            for b in range(2):
                cc = c + b
                wait_in(b)

                @pl.when(cc >= 2)
                def _():
                    wait_out(b)

                compute(b)
                start_out(cc, b)

                @pl.when(cc + 2 < ch_per_w)
                def _():
                    start_in(cc + 2, b)

        wait_out(0)
        wait_out(1)

    return sc_kernel


def kernel(input, coefficients_vect, scaling_coeffs_vect):
    b, c = input.shape
    cs = coefficients_vect.reshape(NUM_ACT, SIZE).T
    cv = pl.pallas_call(
        _project_body,
        out_shape=jax.ShapeDtypeStruct((SIZE, NUM_ACT), jnp.float32),
    )(cs).reshape(-1)
    s = scaling_coeffs_vect.reshape(NUM_ACT)
    pm = s * jnp.float32(1.0 / GRID)
    inv = 1.0 / s
    out = _make_sc_kernel(b * c)(input.reshape(-1), cv, pm, inv)
    return out.reshape(b, c)


# speedup vs baseline: 1.1877x; 1.0541x over previous
import functools

import jax
import jax.numpy as jnp
from jax import lax
from jax.experimental import pallas as pl
from jax.experimental.pallas import tpu as pltpu
from jax.experimental.pallas import tpu_sc as plsc

NUM_ACT = 2048
SIZE = 51
RANGE_ = 4.0
GRID = 2.0 * RANGE_ / (SIZE - 1)
HALF = SIZE // 2

NC, NS, L = 2, 16, 16
NW = NC * NS
NHALF = 2
COLS_W = NUM_ACT // NHALF
TECS_PER_HALF = NW // NHALF
ROWS_PER_CHUNK = 4
CHUNK = ROWS_PER_CHUNK * COLS_W

P_SZ = NUM_ACT * SIZE
Q_SZ = NUM_ACT * (SIZE - 1)
PW = COLS_W * SIZE
QW = COLS_W * (SIZE - 1)
TAB_W = PW + QW

MAGIC_F = 12582912.0
MAGIC_BITS = 0x4B400000


def _project_body(cs_ref, p_ref, q_ref):
    cs = cs_ref[...]
    slopes = jnp.clip(cs[:, 1:] - cs[:, :-1], 0.0, jnp.float32(GRID))
    k = lax.broadcasted_iota(jnp.int32, (SIZE - 1, SIZE), 0)
    j = lax.broadcasted_iota(jnp.int32, (SIZE - 1, SIZE), 1)
    m = (j > k).astype(jnp.float32)
    cum = lax.dot_general(
        slopes, m, (((1,), (0,)), ((), ())),
        preferred_element_type=jnp.float32,
        precision=lax.Precision.HIGHEST)
    cv = cum - cum[:, HALF:HALF + 1]
    qext = jnp.concatenate(
        [slopes, jnp.zeros((NUM_ACT, 1), jnp.float32)], axis=1)
    kk = lax.broadcasted_iota(jnp.int32, (NUM_ACT, SIZE), 1).astype(jnp.float32)
    p_ref[...] = cv - (kk - jnp.float32(0.5)) * qext
    q_ref[...] = slopes


def _make_sc_kernel(n_rows):
    rows_w = n_rows // TECS_PER_HALF
    ch_per_w = rows_w // ROWS_PER_CHUNK
    mesh = plsc.VectorSubcoreMesh(
        core_axis_name="c", subcore_axis_name="s",
        num_cores=NC, num_subcores=NS)

    @functools.partial(
        pl.kernel,
        out_type=jax.ShapeDtypeStruct((n_rows * NUM_ACT,), jnp.float32),
        mesh=mesh,
        compiler_params=pltpu.CompilerParams(needs_layout_passes=False),
        scratch_types=[
            pltpu.VMEM((TAB_W,), jnp.float32),
            pltpu.VMEM((COLS_W,), jnp.float32),
            pltpu.VMEM((COLS_W,), jnp.float32),
            pltpu.VMEM((CHUNK,), jnp.float32),
            pltpu.VMEM((CHUNK,), jnp.float32),
            pltpu.VMEM((CHUNK,), jnp.float32),
            pltpu.VMEM((CHUNK,), jnp.float32),
            pltpu.SemaphoreType.DMA,
            pltpu.SemaphoreType.DMA,
            pltpu.SemaphoreType.DMA,
            pltpu.SemaphoreType.DMA,
        ],
    )
    def sc_kernel(x_hbm, p_hbm, q_hbm, pm_hbm, inv_hbm, out_hbm,
                  table, pm_v, inv_v, xb0, xb1, ob0, ob1,
                  sin0, sin1, sout0, sout1):
        wid = lax.axis_index("s") * NC + lax.axis_index("c")
        h = wid // TECS_PER_HALF
        t = wid % TECS_PER_HALF
        pltpu.sync_copy(p_hbm.at[pl.ds(h * PW, PW)], table.at[pl.ds(0, PW)])
        pltpu.sync_copy(q_hbm.at[pl.ds(h * QW, QW)], table.at[pl.ds(PW, QW)])
        pltpu.sync_copy(pm_hbm.at[pl.ds(h * COLS_W, COLS_W)], pm_v)
        pltpu.sync_copy(inv_hbm.at[pl.ds(h * COLS_W, COLS_W)], inv_v)
        col0 = h * COLS_W
        row0 = t * rows_w
        xbs = (xb0, xb1)
        obs = (ob0, ob1)
        sins = (sin0, sin1)
        souts = (sout0, sout1)

        def start_in(c, b):
            for j in range(ROWS_PER_CHUNK):
                off = (row0 + c * ROWS_PER_CHUNK + j) * NUM_ACT + col0
                pltpu.async_copy(
                    x_hbm.at[pl.ds(off, COLS_W)],
                    xbs[b].at[pl.ds(j * COLS_W, COLS_W)], sins[b])

        def start_out(c, b):
            for j in range(ROWS_PER_CHUNK):
                off = (row0 + c * ROWS_PER_CHUNK + j) * NUM_ACT + col0
                pltpu.async_copy(
                    obs[b].at[pl.ds(j * COLS_W, COLS_W)],
                    out_hbm.at[pl.ds(off, COLS_W)], souts[b])

        def wait_in(b):
            pltpu.make_async_copy(
                x_hbm.at[pl.ds(0, CHUNK)], xbs[b], sins[b]).wait()

        def wait_out(b):
            pltpu.make_async_copy(
                obs[b], out_hbm.at[pl.ds(0, CHUNK)], souts[b]).wait()

        iota51 = lax.iota(jnp.int32, L) * SIZE
        iota50 = lax.iota(jnp.int32, L) * (SIZE - 1)

        def compute(b):
            xb = xbs[b]
            ob = obs[b]

            @plsc.parallel_loop(0, COLS_W // L, unroll=2)
            def _(i):
                col = i * L
                pm = pm_v[pl.ds(col, L)]
                iv = inv_v[pl.ds(col, L)]
                pbase = iota51 + (col * SIZE - MAGIC_BITS)
                qbase = iota50 + (col * (SIZE - 1) + PW - MAGIC_BITS)
                for r in range(ROWS_PER_CHUNK):
                    o = r * COLS_W + col
                    xv = xb[pl.ds(o, L)]
                    w = xv * pm + jnp.float32(HALF - 0.5)
                    wc = jnp.minimum(jnp.maximum(w, -0.5),
                                     jnp.float32(SIZE - 2))
                    ti = plsc.bitcast(wc + jnp.float32(MAGIC_F), jnp.int32)
                    p = plsc.load_gather(table, [ti + pbase])
                    q = plsc.load_gather(table, [ti + qbase])
                    ob[pl.ds(o, L)] = (p + q * w) * iv

        start_in(0, 0)
        start_in(1, 1)

        @pl.loop(0, ch_per_w, step=2)
        def _(c):
            for b in range(2):
                cc = c + b
                wait_in(b)

                @pl.when(cc >= 2)
                def _():
                    wait_out(b)

                compute(b)
                start_out(cc, b)

                @pl.when(cc + 2 < ch_per_w)
                def _():
                    start_in(cc + 2, b)

        wait_out(0)
        wait_out(1)

    return sc_kernel


def kernel(input, coefficients_vect, scaling_coeffs_vect):
    b, c = input.shape
    cs = coefficients_vect.reshape(NUM_ACT, SIZE)
    p, q = pl.pallas_call(
        _project_body,
        out_shape=[
            jax.ShapeDtypeStruct((NUM_ACT, SIZE), jnp.float32),
            jax.ShapeDtypeStruct((NUM_ACT, SIZE - 1), jnp.float32),
        ],
    )(cs)
    s = scaling_coeffs_vect.reshape(NUM_ACT)
    pm = s * jnp.float32(1.0 / GRID)
    inv = 1.0 / s
    out = _make_sc_kernel(b)(
        input.reshape(-1), p.reshape(-1), q.reshape(-1), pm, inv)
    return out.reshape(b, c)


# speedup vs baseline: 2.1025x; 1.7703x over previous
import functools

import jax
import jax.numpy as jnp
from jax import lax
from jax.experimental import pallas as pl
from jax.experimental.pallas import tpu as pltpu
from jax.experimental.pallas import tpu_sc as plsc

NUM_ACT = 2048
SIZE = 51
RANGE_ = 4.0
GRID = 2.0 * RANGE_ / (SIZE - 1)
HALF = SIZE // 2

NC, NS, L = 2, 16, 16
NW = NC * NS
NHALF = 2
COLS_W = NUM_ACT // NHALF
TECS_PER_HALF = NW // NHALF
ROWS_PER_CHUNK = 4
CHUNK = ROWS_PER_CHUNK * COLS_W

P_SZ = NUM_ACT * SIZE
Q_SZ = NUM_ACT * (SIZE - 1)
PW = COLS_W * SIZE
QW = COLS_W * (SIZE - 1)
TAB_W = PW + QW

MAGIC_F = 12582912.0
MAGIC_BITS = 0x4B400000


def _project_body(cs_ref, p_ref, q_ref):
    cs = cs_ref[...]
    slopes = jnp.clip(cs[:, 1:] - cs[:, :-1], 0.0, jnp.float32(GRID))
    k = lax.broadcasted_iota(jnp.int32, (SIZE - 1, SIZE), 0)
    j = lax.broadcasted_iota(jnp.int32, (SIZE - 1, SIZE), 1)
    m = (j > k).astype(jnp.float32)
    cum = lax.dot_general(
        slopes, m, (((1,), (0,)), ((), ())),
        preferred_element_type=jnp.float32,
        precision=lax.Precision.HIGHEST)
    cv = cum - cum[:, HALF:HALF + 1]
    qext = jnp.concatenate(
        [slopes, jnp.zeros((NUM_ACT, 1), jnp.float32)], axis=1)
    kk = lax.broadcasted_iota(jnp.int32, (NUM_ACT, SIZE), 1).astype(jnp.float32)
    p_ref[...] = cv - (kk - jnp.float32(0.5)) * qext
    q_ref[...] = slopes


def _make_sc_kernel(n_rows):
    rows_w = n_rows // TECS_PER_HALF
    ch_per_w = rows_w // ROWS_PER_CHUNK
    mesh = plsc.VectorSubcoreMesh(
        core_axis_name="c", subcore_axis_name="s",
        num_cores=NC, num_subcores=NS)

    @functools.partial(
        pl.kernel,
        out_type=jax.ShapeDtypeStruct((n_rows, NUM_ACT), jnp.float32),
        mesh=mesh,
        compiler_params=pltpu.CompilerParams(needs_layout_passes=False),
        scratch_types=[
            pltpu.VMEM((TAB_W,), jnp.float32),
            pltpu.VMEM((COLS_W,), jnp.float32),
            pltpu.VMEM((COLS_W,), jnp.float32),
            pltpu.VMEM((ROWS_PER_CHUNK, COLS_W), jnp.float32),
            pltpu.VMEM((ROWS_PER_CHUNK, COLS_W), jnp.float32),
            pltpu.VMEM((ROWS_PER_CHUNK, COLS_W), jnp.float32),
            pltpu.VMEM((ROWS_PER_CHUNK, COLS_W), jnp.float32),
            pltpu.SemaphoreType.DMA,
            pltpu.SemaphoreType.DMA,
            pltpu.SemaphoreType.DMA,
            pltpu.SemaphoreType.DMA,
        ],
    )
    def sc_kernel(x_hbm, p_hbm, q_hbm, pm_hbm, inv_hbm, out_hbm,
                  table, pm_v, inv_v, xb0, xb1, ob0, ob1,
                  sin0, sin1, sout0, sout1):
        wid = lax.axis_index("s") * NC + lax.axis_index("c")
        h = wid // TECS_PER_HALF
        t = wid % TECS_PER_HALF
        pltpu.sync_copy(p_hbm.at[pl.ds(h * PW, PW)], table.at[pl.ds(0, PW)])
        pltpu.sync_copy(q_hbm.at[pl.ds(h * QW, QW)], table.at[pl.ds(PW, QW)])
        pltpu.sync_copy(pm_hbm.at[pl.ds(h * COLS_W, COLS_W)], pm_v)
        pltpu.sync_copy(inv_hbm.at[pl.ds(h * COLS_W, COLS_W)], inv_v)
        col0 = h * COLS_W
        row0 = t * rows_w
        xbs = (xb0, xb1)
        obs = (ob0, ob1)
        sins = (sin0, sin1)
        souts = (sout0, sout1)

        def start_in(c, b):
            r = row0 + c * ROWS_PER_CHUNK
            pltpu.async_copy(
                x_hbm.at[pl.ds(r, ROWS_PER_CHUNK), pl.ds(col0, COLS_W)],
                xbs[b], sins[b])

        def start_out(c, b):
            r = row0 + c * ROWS_PER_CHUNK
            pltpu.async_copy(
                obs[b],
                out_hbm.at[pl.ds(r, ROWS_PER_CHUNK), pl.ds(col0, COLS_W)],
                souts[b])

        def wait_in(b):
            pltpu.make_async_copy(
                x_hbm.at[pl.ds(0, ROWS_PER_CHUNK), pl.ds(0, COLS_W)],
                xbs[b], sins[b]).wait()

        def wait_out(b):
            pltpu.make_async_copy(
                obs[b],
                out_hbm.at[pl.ds(0, ROWS_PER_CHUNK), pl.ds(0, COLS_W)],
                souts[b]).wait()

        iota51 = lax.iota(jnp.int32, L) * SIZE
        iota50 = lax.iota(jnp.int32, L) * (SIZE - 1)

        def compute(b):
            xb = xbs[b]
            ob = obs[b]

            @plsc.parallel_loop(0, COLS_W // L, unroll=2)
            def _(i):
                col = i * L
                pm = pm_v[pl.ds(col, L)]
                iv = inv_v[pl.ds(col, L)]
                pbase = iota51 + (col * SIZE - MAGIC_BITS)
                qbase = iota50 + (col * (SIZE - 1) + PW - MAGIC_BITS)
                for r in range(ROWS_PER_CHUNK):
                    xv = xb[r, pl.ds(col, L)]
                    w = xv * pm + jnp.float32(HALF - 0.5)
                    wc = jnp.minimum(jnp.maximum(w, -0.5),
                                     jnp.float32(SIZE - 2))
                    ti = plsc.bitcast(wc + jnp.float32(MAGIC_F), jnp.int32)
                    p = plsc.load_gather(table, [ti + pbase])
                    q = plsc.load_gather(table, [ti + qbase])
                    ob[r, pl.ds(col, L)] = (p + q * w) * iv

        start_in(0, 0)
        start_in(1, 1)

        @pl.loop(0, ch_per_w, step=2)
        def _(c):
            for b in range(2):
                cc = c + b
                wait_in(b)

                @pl.when(cc >= 2)
                def _():
                    wait_out(b)

                compute(b)
                start_out(cc, b)

                @pl.when(cc + 2 < ch_per_w)
                def _():
                    start_in(cc + 2, b)

        wait_out(0)
        wait_out(1)

    return sc_kernel


def kernel(input, coefficients_vect, scaling_coeffs_vect):
    b, c = input.shape
    cs = coefficients_vect.reshape(NUM_ACT, SIZE)
    p, q = pl.pallas_call(
        _project_body,
        out_shape=[
            jax.ShapeDtypeStruct((NUM_ACT, SIZE), jnp.float32),
            jax.ShapeDtypeStruct((NUM_ACT, SIZE - 1), jnp.float32),
        ],
    )(cs)
    s = scaling_coeffs_vect.reshape(NUM_ACT)
    pm = s * jnp.float32(1.0 / GRID)
    inv = 1.0 / s
    return _make_sc_kernel(b)(input, p.reshape(-1), q.reshape(-1), pm, inv)
